# baseline (speedup 1.0000x reference)
import jax
import jax.numpy as jnp
from jax import lax
from jax.experimental import pallas as pl
from jax.experimental.pallas import tpu as pltpu
from jax.experimental.pallas import tpu_sc as plsc

B = 16384
D = 128
NF = 9
VF = 100
PSTR = 104
PROWS = VF * PSTR
NP = 4
KIN = 164
BG = 2048

NC = 2
NS = 16
NW = NC * NS
RPW = B // NW
CH = 64
NCHUNK = RPW // CH
NIX = NF
NTG = NP + 1
NBUF = NTG + 1
GPC = CH // 16


def _pair_body(tpair_ref, w_ref, pp_ref):
    pa = lax.dot_general(
        tpair_ref[0, 0], w_ref[0, :, 0:16],
        (((1,), (1,)), ((), ())), preferred_element_type=jnp.float32,
        precision=lax.Precision.HIGHEST)
    pb = lax.dot_general(
        tpair_ref[0, 1], w_ref[0, :, 16:32],
        (((1,), (1,)), ((), ())), preferred_element_type=jnp.float32,
        precision=lax.Precision.HIGHEST)
    for a in range(VF):
        pp_ref[PSTR * a:PSTR * a + VF, :] = pa[a:a + 1, :] + pb


_pair_call = pl.pallas_call(
    _pair_body,
    grid=(NP,),
    in_specs=[
        pl.BlockSpec((1, 2, VF, 16), lambda p: (p, 0, 0, 0)),
        pl.BlockSpec((1, D, 32), lambda p: (p, 0, 0)),
    ],
    out_specs=pl.BlockSpec((PROWS, D), lambda p: (p, 0)),
    out_shape=jax.ShapeDtypeStruct((NP * PROWS, D), jnp.float32),
)


def _gp_body(tmp_ref, genre_ref, t8_ref, w_ref, b_ref, o_ref):
    p8 = lax.dot_general(
        t8_ref[...], w_ref[:, 128:144],
        (((1,), (1,)), ((), ())), preferred_element_type=jnp.float32,
        precision=lax.Precision.HIGHEST).astype(jnp.bfloat16)
    iota = lax.broadcasted_iota(jnp.int32, (BG, VF), 1)
    oh = (tmp_ref[...] == iota).astype(jnp.bfloat16)
    m8 = lax.dot_general(
        oh, p8, (((1,), (0,)), ((), ())),
        preferred_element_type=jnp.float32)
    wg = w_ref[:, 144:KIN]
    o_ref[...] = (m8 + b_ref[...]
                  + lax.dot_general(
                      genre_ref[...], wg, (((1,), (1,)), ((), ())),
                      preferred_element_type=jnp.float32,
                      precision=lax.Precision.HIGHEST))


_gp_call = pl.pallas_call(
    _gp_body,
    grid=(B // BG,),
    in_specs=[
        pl.BlockSpec((BG, 1), lambda i: (i, 0)),
        pl.BlockSpec((BG, 20), lambda i: (i, 0)),
        pl.BlockSpec((VF, 16), lambda i: (0, 0)),
        pl.BlockSpec((D, KIN), lambda i: (0, 0)),
        pl.BlockSpec((1, D), lambda i: (0, 0)),
    ],
    out_specs=pl.BlockSpec((BG, D), lambda i: (i, 0)),
    out_shape=jax.ShapeDtypeStruct((B, D), jnp.float32),
)


def _sc_body(pp_hbm, tid_hbm, g_hbm, idx_hbm, s_hbm, st, ib, *rest):
    bufs = [rest[par * NBUF:(par + 1) * NBUF] for par in range(2)]
    gsem = rest[2 * NBUF:2 * NBUF + 2]
    wsem = rest[2 * NBUF + 2:2 * NBUF + 4]
    wid = lax.axis_index("s") * NC + lax.axis_index("c")
    base = wid * RPW

    icopies = [pltpu.async_copy(idx_hbm.at[t, pl.ds(base, RPW)],
                                st.at[t], gsem[0]) for t in range(NIX)]
    for cpy in icopies:
        cpy.wait()
    for g in range(RPW // 16):
        k = g // GPC
        o = pl.ds((g % GPC) * 16, 16)
        s = pl.ds(16 * g, 16)
        for p in range(NP):
            ib[p, k, o] = (st[2 * p, s] * PSTR + st[2 * p + 1, s]
                           + p * PROWS)
        ib[NP, k, o] = st[NF - 1, s]

    def fire(k, par):
        cps = [pltpu.async_copy(pp_hbm.at[ib.at[p, k]], bufs[par][p],
                                gsem[par]) for p in range(NP)]
        cps.append(pltpu.async_copy(tid_hbm.at[ib.at[NP, k]],
                                    bufs[par][NP], gsem[par]))
        cps.append(pltpu.async_copy(g_hbm.at[pl.ds(base + k * CH, CH)],
                                    bufs[par][NTG], gsem[par]))
        return cps

    pend = {0: fire(0, 0)}
    wr = [None, None]
    for k in range(NCHUNK):
        par = k & 1
        npar = 1 - par
        if k + 1 < NCHUNK:
            if wr[npar] is not None:
                wr[npar].wait()
            pend[k + 1] = fire(k + 1, npar)
        for cpy in pend.pop(k):
            cpy.wait()

        def row_body(r, carry):
            for j in range(D // 16):
                s = pl.ds(16 * j, 16)
                v = bufs[par][0][r, s]
                for t in range(1, NBUF):
                    v = v + bufs[par][t][r, s]
                bufs[par][0][r, s] = v
            return carry
        lax.fori_loop(0, CH, row_body, 0)
        wr[par] = pltpu.async_copy(
            bufs[par][0], s_hbm.at[pl.ds(base + k * CH, CH)], wsem[par])
    wr[0].wait()
    wr[1].wait()


_sc_call = pl.kernel(
    _sc_body,
    mesh=plsc.VectorSubcoreMesh(core_axis_name="c", subcore_axis_name="s"),
    out_type=jax.ShapeDtypeStruct((B, D), jnp.float32),
    scratch_types=(
        [pltpu.VMEM((NF + 1, RPW), jnp.int32),
         pltpu.VMEM((NTG, NCHUNK, CH), jnp.int32)]
        + [pltpu.VMEM((CH, D), jnp.float32)] * (2 * NBUF)
        + [pltpu.SemaphoreType.DMA] * 4
    ),
)


def kernel(danceability, energy, loudness, speechiness, acousticness,
           instrumentalness, liveness, valence, tempo,
           table_danceability, table_energy, table_loudness,
           table_speechiness, table_acousticness, table_instrumentalness,
           table_liveness, table_valence, table_tempo,
           id, table_id, genre, W, b):
    idx = jnp.stack([danceability, energy, loudness, speechiness,
                     acousticness, instrumentalness, liveness, valence,
                     id, id]).astype(jnp.int32)
    tpairs = jnp.stack([table_danceability, table_energy, table_loudness,
                        table_speechiness, table_acousticness,
                        table_instrumentalness, table_liveness,
                        table_valence]).reshape(NP, 2, VF, 16)
    wpairs = W[:, :128].reshape(D, NP, 32).transpose(1, 0, 2)
    pp = _pair_call(tpairs, wpairs)
    tmp32 = tempo.astype(jnp.int32).reshape(B, 1)
    g = _gp_call(tmp32, genre, table_tempo, W, b.reshape(1, D))
    return _sc_call(pp, table_id, g, idx)


# speedup vs baseline: 1.1523x; 1.1523x over previous
import jax
import jax.numpy as jnp
from jax import lax
from jax.experimental import pallas as pl
from jax.experimental.pallas import tpu as pltpu
from jax.experimental.pallas import tpu_sc as plsc

B = 16384
D = 128
NF = 9
VF = 100
PSTR = 104
PROWS = VF * PSTR
NP = 4
KIN = 164
BG = 2048

NC = 2
NS = 16
NW = NC * NS
RPW = B // NW
CH = 64
NCHUNK = RPW // CH
NIX = NF
NTG = NP + 1
GPC = CH // 16


def _pair_body(tpair_ref, w_ref, pp_ref):
    pa = lax.dot_general(
        tpair_ref[0, 0], w_ref[0, :, 0:16],
        (((1,), (1,)), ((), ())), preferred_element_type=jnp.float32,
        precision=lax.Precision.HIGHEST)
    pb = lax.dot_general(
        tpair_ref[0, 1], w_ref[0, :, 16:32],
        (((1,), (1,)), ((), ())), preferred_element_type=jnp.float32,
        precision=lax.Precision.HIGHEST)
    for a in range(VF):
        pp_ref[PSTR * a:PSTR * a + VF, :] = pa[a:a + 1, :] + pb


_pair_call = pl.pallas_call(
    _pair_body,
    grid=(NP,),
    in_specs=[
        pl.BlockSpec((1, 2, VF, 16), lambda p: (p, 0, 0, 0)),
        pl.BlockSpec((1, D, 32), lambda p: (p, 0, 0)),
    ],
    out_specs=pl.BlockSpec((PROWS, D), lambda p: (p, 0)),
    out_shape=jax.ShapeDtypeStruct((NP * PROWS, D), jnp.float32),
)


def _fin_body(s_ref, tmp_ref, genre_ref, t8_ref, w_ref, b_ref, o_ref):
    p8 = lax.dot_general(
        t8_ref[...], w_ref[:, 128:144],
        (((1,), (1,)), ((), ())), preferred_element_type=jnp.float32,
        precision=lax.Precision.HIGHEST).astype(jnp.bfloat16)
    iota = lax.broadcasted_iota(jnp.int32, (BG, VF), 1)
    oh = (tmp_ref[...] == iota).astype(jnp.bfloat16)
    m8 = lax.dot_general(
        oh, p8, (((1,), (0,)), ((), ())),
        preferred_element_type=jnp.float32)
    wg = w_ref[:, 144:KIN]
    o_ref[...] = (s_ref[...] + m8 + b_ref[...]
                  + lax.dot_general(
                      genre_ref[...], wg, (((1,), (1,)), ((), ())),
                      preferred_element_type=jnp.float32,
                      precision=lax.Precision.HIGHEST))


_fin_call = pl.pallas_call(
    _fin_body,
    grid=(B // BG,),
    in_specs=[
        pl.BlockSpec((BG, D), lambda i: (i, 0)),
        pl.BlockSpec((BG, 1), lambda i: (i, 0)),
        pl.BlockSpec((BG, 20), lambda i: (i, 0)),
        pl.BlockSpec((VF, 16), lambda i: (0, 0)),
        pl.BlockSpec((D, KIN), lambda i: (0, 0)),
        pl.BlockSpec((1, D), lambda i: (0, 0)),
    ],
    out_specs=pl.BlockSpec((BG, D), lambda i: (i, 0)),
    out_shape=jax.ShapeDtypeStruct((B, D), jnp.float32),
)


def _sc_body(pp_hbm, tid_hbm, idx_hbm, s_hbm, st, ib, *rest):
    bufs = [rest[par * NTG:(par + 1) * NTG] for par in range(2)]
    gsem = rest[2 * NTG:2 * NTG + 2]
    wsem = rest[2 * NTG + 2:2 * NTG + 4]
    wid = lax.axis_index("s") * NC + lax.axis_index("c")
    base = wid * RPW

    icopies = [pltpu.async_copy(idx_hbm.at[t, pl.ds(base, RPW)],
                                st.at[t], gsem[0]) for t in range(NIX)]
    for cpy in icopies:
        cpy.wait()
    for g in range(RPW // 16):
        k = g // GPC
        o = pl.ds((g % GPC) * 16, 16)
        s = pl.ds(16 * g, 16)
        for p in range(NP):
            ib[p, k, o] = (st[2 * p, s] * PSTR + st[2 * p + 1, s]
                           + p * PROWS)
        ib[NP, k, o] = st[NF - 1, s]

    def fire(k, par):
        cps = [pltpu.async_copy(pp_hbm.at[ib.at[p, k]], bufs[par][p],
                                gsem[par]) for p in range(NP)]
        cps.append(pltpu.async_copy(tid_hbm.at[ib.at[NP, k]],
                                    bufs[par][NP], gsem[par]))
        return cps

    pend = {0: fire(0, 0)}
    wr = [None, None]
    for k in range(NCHUNK):
        par = k & 1
        npar = 1 - par
        if k + 1 < NCHUNK:
            if wr[npar] is not None:
                wr[npar].wait()
            pend[k + 1] = fire(k + 1, npar)
        for cpy in pend.pop(k):
            cpy.wait()

        def row_body(r, carry):
            for j in range(D // 16):
                s = pl.ds(16 * j, 16)
                v = bufs[par][0][r, s]
                for t in range(1, NTG):
                    v = v + bufs[par][t][r, s]
                bufs[par][0][r, s] = v
            return carry
        lax.fori_loop(0, CH, row_body, 0)
        wr[par] = pltpu.async_copy(
            bufs[par][0], s_hbm.at[pl.ds(base + k * CH, CH)], wsem[par])
    wr[0].wait()
    wr[1].wait()


_sc_call = pl.kernel(
    _sc_body,
    mesh=plsc.VectorSubcoreMesh(core_axis_name="c", subcore_axis_name="s"),
    out_type=jax.ShapeDtypeStruct((B, D), jnp.float32),
    scratch_types=(
        [pltpu.VMEM((NF + 1, RPW), jnp.int32),
         pltpu.VMEM((NTG, NCHUNK, CH), jnp.int32)]
        + [pltpu.VMEM((CH, D), jnp.float32)] * (2 * NTG)
        + [pltpu.SemaphoreType.DMA] * 4
    ),
)


def kernel(danceability, energy, loudness, speechiness, acousticness,
           instrumentalness, liveness, valence, tempo,
           table_danceability, table_energy, table_loudness,
           table_speechiness, table_acousticness, table_instrumentalness,
           table_liveness, table_valence, table_tempo,
           id, table_id, genre, W, b):
    idx = jnp.stack([danceability, energy, loudness, speechiness,
                     acousticness, instrumentalness, liveness, valence,
                     id, id]).astype(jnp.int32)
    tpairs = jnp.stack([table_danceability, table_energy, table_loudness,
                        table_speechiness, table_acousticness,
                        table_instrumentalness, table_liveness,
                        table_valence]).reshape(NP, 2, VF, 16)
    wpairs = W[:, :128].reshape(D, NP, 32).transpose(1, 0, 2)
    pp = _pair_call(tpairs, wpairs)
    s = _sc_call(pp, table_id, idx)
    tmp32 = tempo.astype(jnp.int32).reshape(B, 1)
    return _fin_call(s, tmp32, genre, table_tempo, W, b.reshape(1, D))
